# baseline (speedup 1.0000x reference)
import functools

import jax
import jax.numpy as jnp
import numpy as np
from jax import lax
from jax.experimental import pallas as pl
from jax.experimental.pallas import tpu as pltpu
from jax.experimental.pallas import tpu_sc as plsc

B, L, C, F = 1024, 20, 26, 6
VOCAB, D, LFREQ = 100000, 32, 8
NT = B * L
NFIELD = C + F

NC, NS, LANES = 2, 16, 16
NW = NC * NS
TCHUNK = 5120
NCHUNK = NT // TCHUNK
PSPLIT = 25088


def _cont_tc_kernel(cont_ref, ws_ref, wc_ref, mask_ref, freqs_ref, out_ref):
    freqs = freqs_ref[0]
    for i in range(F):
        v = cont_ref[i]
        m = jnp.isnan(v)
        clean = jnp.where(m, jnp.zeros_like(v), v)
        x = freqs[:, None] * clean[None, :]
        y = (jnp.dot(ws_ref[i], jnp.sin(x), preferred_element_type=jnp.float32)
             + jnp.dot(wc_ref[i], jnp.cos(x), preferred_element_type=jnp.float32))
        out_ref[i] = jnp.where(m[None, :], mask_ref[0][:, None], y)


def _cont_embs(cont2, W_proj, mask_token):
    Ws = W_proj[:, :, 0::2]
    Wc = W_proj[:, :, 1::2]
    return pl.pallas_call(
        _cont_tc_kernel,
        out_shape=jax.ShapeDtypeStruct((F, D, NT), jnp.float32),
    )(cont2, Ws, Wc, mask_token[None],
      jnp.asarray((2.0 ** np.arange(LFREQ)) * np.pi, jnp.float32)[None])


VSPLIT = 50048
VREST = VOCAB - VSPLIT


def _sc_body(tabs_hbm, cat2_hbm, conty_hbm, out_hbm,
             ha, hb, outpl, idx0, idx1, so, si0, si1, sa, sb):
    d = lax.axis_index("s") * NC + lax.axis_index("c")

    def gather_chunk(base, idxb, half, is_b):
        if not is_b:
            @plsc.parallel_loop(0, TCHUNK, LANES, unroll=8)
            def _(i):
                iv = idxb[pl.ds(i, LANES)]
                outpl[pl.ds(base + i, LANES)] = plsc.load_gather(
                    half, [jnp.minimum(iv, VSPLIT - 1)])
        else:
            @plsc.parallel_loop(0, TCHUNK, LANES, unroll=8)
            def _(i):
                iv = idxb[pl.ds(i, LANES)]
                osl = pl.ds(base + i, LANES)
                g = plsc.load_gather(half, [jnp.maximum(iv - VSPLIT, 0)])
                outpl[osl] = jnp.where(iv >= VSPLIT, g, outpl[osl])

    def gather_pass(k, half, is_b):
        pltpu.async_copy(cat2_hbm.at[k, pl.ds(0, TCHUNK)], idx0, si0)

        def pair(j, _):
            base0 = 2 * j * TCHUNK
            pltpu.make_async_copy(
                cat2_hbm.at[k, pl.ds(0, TCHUNK)], idx0, si0).wait()
            pltpu.async_copy(
                cat2_hbm.at[k, pl.ds(base0 + TCHUNK, TCHUNK)], idx1, si1)
            gather_chunk(base0, idx0, half, is_b)
            pltpu.make_async_copy(
                cat2_hbm.at[k, pl.ds(0, TCHUNK)], idx1, si1).wait()

            @pl.when(j + 1 < NCHUNK // 2)
            def _():
                pltpu.async_copy(
                    cat2_hbm.at[k, pl.ds(base0 + 2 * TCHUNK, TCHUNK)],
                    idx0, si0)

            gather_chunk(base0 + TCHUNK, idx1, half, is_b)
            return 0

        lax.fori_loop(0, NCHUNK // 2, pair, 0)

    def do_plane(k, _):
        @pl.when(k > 0)
        def _():
            pltpu.make_async_copy(outpl, out_hbm.at[0, d, :], so).wait()

        @pl.when(k < C)
        def _():
            pltpu.async_copy(tabs_hbm.at[k, d, pl.ds(VSPLIT, VREST)], hb, sb)
            pltpu.make_async_copy(
                tabs_hbm.at[k, d, pl.ds(0, VSPLIT)], ha, sa).wait()
            gather_pass(k, ha, is_b=False)

            @pl.when(k + 1 < C)
            def _():
                pltpu.async_copy(
                    tabs_hbm.at[k + 1, d, pl.ds(0, VSPLIT)], ha, sa)

            pltpu.make_async_copy(
                tabs_hbm.at[k, d, pl.ds(VSPLIT, VREST)], hb, sb).wait()
            gather_pass(k, hb, is_b=True)

        @pl.when(k >= C)
        def _():
            pltpu.sync_copy(conty_hbm.at[k - C, d, :], outpl)

        pltpu.async_copy(outpl, out_hbm.at[k, d, :], so)
        return 0

    pltpu.async_copy(tabs_hbm.at[0, d, pl.ds(0, VSPLIT)], ha, sa)
    lax.fori_loop(0, NFIELD, do_plane, 0)
    pltpu.make_async_copy(outpl, out_hbm.at[0, d, :], so).wait()


@jax.jit
def _run(tables_t, cat2, cont_y):
    mesh = plsc.VectorSubcoreMesh(core_axis_name="c", subcore_axis_name="s")
    sc = pl.kernel(
        _sc_body,
        out_type=jax.ShapeDtypeStruct((NFIELD, D, NT), jnp.float32),
        mesh=mesh,
        compiler_params=pltpu.CompilerParams(needs_layout_passes=False),
        scratch_types=[
            pltpu.VMEM((VSPLIT,), jnp.float32),
            pltpu.VMEM((VREST,), jnp.float32),
            pltpu.VMEM((NT,), jnp.float32),
            pltpu.VMEM((TCHUNK,), jnp.int32),
            pltpu.VMEM((TCHUNK,), jnp.int32),
            pltpu.SemaphoreType.DMA,
            pltpu.SemaphoreType.DMA,
            pltpu.SemaphoreType.DMA,
            pltpu.SemaphoreType.DMA,
            pltpu.SemaphoreType.DMA,
        ],
    )
    return sc(tables_t, cat2, cont_y)


def kernel(cat, cont, tables, W_proj, mask_token):
    tables_t = jnp.transpose(tables, (0, 2, 1))
    cat2 = jnp.transpose(cat.astype(jnp.int32).reshape(NT, C), (1, 0))
    cont_y = _cont_embs(
        jnp.transpose(cont.reshape(NT, F), (1, 0)), W_proj, mask_token)
    out_t = _run(tables_t, cat2, cont_y)
    return jnp.transpose(out_t, (2, 0, 1))


# speedup vs baseline: 1.2712x; 1.2712x over previous
import functools

import jax
import jax.numpy as jnp
import numpy as np
from jax import lax
from jax.experimental import pallas as pl
from jax.experimental.pallas import tpu as pltpu
from jax.experimental.pallas import tpu_sc as plsc

B, L, C, F = 1024, 20, 26, 6
VOCAB, D, LFREQ = 100000, 32, 8
NT = B * L
NFIELD = C + F

NC, NS, LANES = 2, 16, 16
NW = NC * NS
TCHUNK = 5120
NCHUNK = NT // TCHUNK
PSPLIT = 25088


def _cont_tc_kernel(cont_ref, ws_ref, wc_ref, mask_ref, freqs_ref, out_ref):
    freqs = freqs_ref[0]
    for i in range(F):
        v = cont_ref[i]
        m = jnp.isnan(v)
        clean = jnp.where(m, jnp.zeros_like(v), v)
        x = freqs[:, None] * clean[None, :]
        y = (jnp.dot(ws_ref[i], jnp.sin(x), preferred_element_type=jnp.float32)
             + jnp.dot(wc_ref[i], jnp.cos(x), preferred_element_type=jnp.float32))
        out_ref[i] = jnp.where(m[None, :], mask_ref[0][:, None], y)


def _cont_embs(cont2, W_proj, mask_token):
    Ws = W_proj[:, :, 0::2]
    Wc = W_proj[:, :, 1::2]
    return pl.pallas_call(
        _cont_tc_kernel,
        out_shape=jax.ShapeDtypeStruct((F, D, NT), jnp.float32),
    )(cont2, Ws, Wc, mask_token[None],
      jnp.asarray((2.0 ** np.arange(LFREQ)) * np.pi, jnp.float32)[None])


def _sc_body(tabs_hbm, cat2_hbm, conty_hbm, out_hbm,
             plane, outpl, idx0, idx1, so, si0, si1, sp):
    d = lax.axis_index("s") * NC + lax.axis_index("c")
    psizes = [PSPLIT, PSPLIT, PSPLIT, VOCAB - 3 * PSPLIT]

    def gather_chunk(base, idxb):
        @plsc.parallel_loop(0, TCHUNK, LANES, unroll=16)
        def _(i):
            outpl[pl.ds(base + i, LANES)] = plsc.load_gather(
                plane, [idxb[pl.ds(i, LANES)]])

    def do_plane(k, _):
        @pl.when(k > 0)
        def _():
            pltpu.make_async_copy(outpl, out_hbm.at[0, d, :], so).wait()

        @pl.when(k < C)
        def _():
            pltpu.async_copy(cat2_hbm.at[k, pl.ds(0, TCHUNK)], idx0, si0)
            pltpu.sync_copy(tabs_hbm.at[k, d, :], plane)

            def pair(j, _):
                base0 = 2 * j * TCHUNK
                pltpu.make_async_copy(
                    cat2_hbm.at[k, pl.ds(0, TCHUNK)], idx0, si0).wait()
                pltpu.async_copy(
                    cat2_hbm.at[k, pl.ds(base0 + TCHUNK, TCHUNK)], idx1, si1)
                gather_chunk(base0, idx0)
                pltpu.make_async_copy(
                    cat2_hbm.at[k, pl.ds(0, TCHUNK)], idx1, si1).wait()

                @pl.when(j + 1 < NCHUNK // 2)
                def _():
                    pltpu.async_copy(
                        cat2_hbm.at[k, pl.ds(base0 + 2 * TCHUNK, TCHUNK)],
                        idx0, si0)

                gather_chunk(base0 + TCHUNK, idx1)
                return 0

            lax.fori_loop(0, NCHUNK // 2, pair, 0)

        @pl.when(k >= C)
        def _():
            pltpu.sync_copy(conty_hbm.at[k - C, d, :], outpl)

        pltpu.async_copy(outpl, out_hbm.at[k, d, :], so)
        return 0

    lax.fori_loop(0, NFIELD, do_plane, 0)
    pltpu.make_async_copy(outpl, out_hbm.at[0, d, :], so).wait()


@jax.jit
def _run(tables_t, cat2, cont_y):
    mesh = plsc.VectorSubcoreMesh(core_axis_name="c", subcore_axis_name="s")
    sc = pl.kernel(
        _sc_body,
        out_type=jax.ShapeDtypeStruct((NFIELD, D, NT), jnp.float32),
        mesh=mesh,
        compiler_params=pltpu.CompilerParams(needs_layout_passes=False),
        scratch_types=[
            pltpu.VMEM((VOCAB,), jnp.float32),
            pltpu.VMEM((NT,), jnp.float32),
            pltpu.VMEM((TCHUNK,), jnp.int32),
            pltpu.VMEM((TCHUNK,), jnp.int32),
            pltpu.SemaphoreType.DMA,
            pltpu.SemaphoreType.DMA,
            pltpu.SemaphoreType.DMA,
            pltpu.SemaphoreType.DMA,
        ],
    )
    return sc(tables_t, cat2, cont_y)


def kernel(cat, cont, tables, W_proj, mask_token):
    tables_t = jnp.transpose(tables, (0, 2, 1))
    cat2 = jnp.transpose(cat.astype(jnp.int32).reshape(NT, C), (1, 0))
    cont_y = _cont_embs(
        jnp.transpose(cont.reshape(NT, F), (1, 0)), W_proj, mask_token)
    out_t = _run(tables_t, cat2, cont_y)
    return jnp.transpose(out_t, (2, 0, 1))
